# Initial kernel scaffold; baseline (speedup 1.0000x reference)
import jax
import jax.numpy as jnp
from jax.experimental import pallas as pl


def kernel(input_tensor, token_emb, segment_emb, ln_w, ln_b):
    raise NotImplementedError("write your pallas kernel here")


# speedup vs baseline: 6.0272x; 6.0272x over previous
import functools
import math

import jax
import jax.numpy as jnp
from jax import lax
from jax.experimental import pallas as pl
from jax.experimental.pallas import tpu as pltpu
from jax.experimental.pallas import tpu_sc as plsc

_NC, _NS = 2, 16
_NW = _NC * _NS
_CHUNK = 128


def _sc_gather(token_emb, idx3):
    nw, n_chunks, ch = idx3.shape
    rows_total = nw * n_chunks * ch
    dim = token_emb.shape[1]
    mesh = plsc.VectorSubcoreMesh(core_axis_name="c", subcore_axis_name="s")

    @functools.partial(
        pl.kernel,
        mesh=mesh,
        out_type=jax.ShapeDtypeStruct((rows_total, dim), jnp.float32),
        scratch_types=[
            pltpu.VMEM((n_chunks, ch), jnp.int32),
            pltpu.VMEM((ch, dim), jnp.float32),
            pltpu.SemaphoreType.DMA,
        ],
    )
    def gather_kernel(table_hbm, idx_hbm, out_hbm, idx_v, rows_v, sem):
        wid = lax.axis_index("s") * _NC + lax.axis_index("c")
        pltpu.sync_copy(idx_hbm.at[wid], idx_v)
        base = wid * (n_chunks * ch)

        def body(j, carry):
            pltpu.async_copy(table_hbm.at[idx_v.at[j]], rows_v, sem).wait()
            pltpu.sync_copy(rows_v, out_hbm.at[pl.ds(base + j * ch, ch)])
            return carry

        lax.fori_loop(0, n_chunks, body, 0)

    return gather_kernel(token_emb, idx3)


def _ln_body(x_ref, seg_ref, w_ref, b_ref, out_ref, const_ref):
    sent_len, dim = const_ref.shape

    @pl.when(pl.program_id(0) == 0)
    def _():
        row = lax.broadcasted_iota(jnp.int32, (sent_len, dim), 0)
        col = lax.broadcasted_iota(jnp.int32, (sent_len, dim), 1)
        pos = row.astype(jnp.float32)
        dexp = 2.0 * col.astype(jnp.float32) * (1.0 / dim)
        angle = pos * jnp.exp(-math.log(10000.0) * dexp)
        pe = jnp.where(col % 2 == 0, jnp.sin(angle), jnp.cos(angle))
        segc = jnp.where(row >= (sent_len // 2 + 1),
                         seg_ref[1, :][None, :], seg_ref[0, :][None, :])
        const_ref[...] = pe + segc

    y = x_ref[...] + const_ref[...][None]
    mean = jnp.mean(y, axis=-1, keepdims=True)
    yc = y - mean
    var = jnp.mean(yc * yc, axis=-1, keepdims=True)
    inv = lax.rsqrt(var + 1e-5)
    out_ref[...] = yc * inv * w_ref[...] + b_ref[...]


def _tc_ln(tok, segment_emb, ln_w, ln_b):
    bsz, sent_len, dim = tok.shape
    bb = 8
    return pl.pallas_call(
        _ln_body,
        grid=(bsz // bb,),
        in_specs=[
            pl.BlockSpec((bb, sent_len, dim), lambda i: (i, 0, 0)),
            pl.BlockSpec((8, dim), lambda i: (0, 0)),
            pl.BlockSpec((1, dim), lambda i: (0, 0)),
            pl.BlockSpec((1, dim), lambda i: (0, 0)),
        ],
        out_specs=pl.BlockSpec((bb, sent_len, dim), lambda i: (i, 0, 0)),
        out_shape=jax.ShapeDtypeStruct((bsz, sent_len, dim), jnp.float32),
        scratch_shapes=[pltpu.VMEM((sent_len, dim), jnp.float32)],
    )(tok, segment_emb, ln_w.reshape(1, dim), ln_b.reshape(1, dim))


def kernel(input_tensor, token_emb, segment_emb, ln_w, ln_b):
    bsz, sent_len = input_tensor.shape
    dim = token_emb.shape[1]
    n_rows = bsz * sent_len
    n_chunks = n_rows // (_NW * _CHUNK)
    idx3 = input_tensor.astype(jnp.int32).reshape(_NW, n_chunks, _CHUNK)
    tok = _sc_gather(token_emb, idx3).reshape(bsz, sent_len, dim)
    return _tc_ln(tok, segment_emb, ln_w, ln_b)


# speedup vs baseline: 6.9222x; 1.1485x over previous
import functools
import math

import jax
import jax.numpy as jnp
from jax import lax
from jax.experimental import pallas as pl
from jax.experimental.pallas import tpu as pltpu
from jax.experimental.pallas import tpu_sc as plsc

_NC, _NS = 2, 16
_NW = _NC * _NS
_K = 8
_CHUNK = 80


def _sc_gather(token_emb, idx3):
    nw, n_chunks, ch = idx3.shape
    rows_total = nw * n_chunks * ch
    dim = token_emb.shape[1]
    mesh = plsc.VectorSubcoreMesh(core_axis_name="c", subcore_axis_name="s")

    @functools.partial(
        pl.kernel,
        mesh=mesh,
        out_type=jax.ShapeDtypeStruct((rows_total, dim), jnp.float32),
        scratch_types=[
            pltpu.VMEM((n_chunks, ch), jnp.int32),
            pltpu.VMEM((ch, dim), jnp.float32),
            pltpu.SemaphoreType.DMA,
        ],
    )
    def gather_kernel(table_hbm, idx_hbm, out_hbm, idx_v, rows_v, sem):
        wid = lax.axis_index("s") * _NC + lax.axis_index("c")
        pltpu.sync_copy(idx_hbm.at[wid], idx_v)
        base = wid * (n_chunks * ch)

        def body(j, carry):
            pltpu.async_copy(table_hbm.at[idx_v.at[j]], rows_v, sem).wait()
            pltpu.sync_copy(rows_v, out_hbm.at[pl.ds(base + j * ch, ch)])
            return carry

        lax.fori_loop(0, n_chunks, body, 0)

    return gather_kernel(token_emb, idx3)


def _make_const(const_ref, seg_ref):
    sent_len, dim = const_ref.shape
    row = lax.broadcasted_iota(jnp.int32, (sent_len, dim), 0)
    col = lax.broadcasted_iota(jnp.int32, (sent_len, dim), 1)
    pos = row.astype(jnp.float32)
    dexp = 2.0 * col.astype(jnp.float32) * (1.0 / dim)
    angle = pos * jnp.exp(-math.log(10000.0) * dexp)
    pe = jnp.where(col % 2 == 0, jnp.sin(angle), jnp.cos(angle))
    segc = jnp.where(row >= (sent_len // 2 + 1),
                     seg_ref[1, :][None, :], seg_ref[0, :][None, :])
    const_ref[...] = pe + segc


def _ln_compute(x_ref, w_ref, b_ref, out_ref, const_ref):
    y = x_ref[...] + const_ref[...][None]
    mean = jnp.mean(y, axis=-1, keepdims=True)
    yc = y - mean
    var = jnp.mean(yc * yc, axis=-1, keepdims=True)
    inv = lax.rsqrt(var + 1e-5)
    out_ref[...] = yc * inv * w_ref[...] + b_ref[...]


def _ln_body_first(x_ref, seg_ref, w_ref, b_ref, out_ref, const_ref):
    @pl.when(pl.program_id(0) == 0)
    def _():
        _make_const(const_ref, seg_ref)

    _ln_compute(x_ref, w_ref, b_ref, out_ref, const_ref)


def _ln_body_chained(buf_ref, x_ref, seg_ref, w_ref, b_ref, out_ref, const_ref):
    del buf_ref
    @pl.when(pl.program_id(0) == 0)
    def _():
        _make_const(const_ref, seg_ref)

    _ln_compute(x_ref, w_ref, b_ref, out_ref, const_ref)


def _tc_ln_chunk(buf, tok, segment_emb, ln_w1, ln_b1, chunk, bsz_total):
    bsz_c, sent_len, dim = tok.shape
    bb = 8
    nsteps = bsz_c // bb
    base = chunk * nsteps
    x_spec = pl.BlockSpec((bb, sent_len, dim), lambda i: (i, 0, 0))
    small_specs = [
        pl.BlockSpec((8, dim), lambda i: (0, 0)),
        pl.BlockSpec((1, dim), lambda i: (0, 0)),
        pl.BlockSpec((1, dim), lambda i: (0, 0)),
    ]
    out_spec = pl.BlockSpec((bb, sent_len, dim), lambda i: (base + i, 0, 0))
    out_shape = jax.ShapeDtypeStruct((bsz_total, sent_len, dim), jnp.float32)
    scratch = [pltpu.VMEM((sent_len, dim), jnp.float32)]
    if buf is None:
        return pl.pallas_call(
            _ln_body_first,
            grid=(nsteps,),
            in_specs=[x_spec] + small_specs,
            out_specs=out_spec,
            out_shape=out_shape,
            scratch_shapes=scratch,
        )(tok, segment_emb, ln_w1, ln_b1)
    buf_spec = pl.BlockSpec((1, 8, dim), lambda i: (0, 0, 0))
    return pl.pallas_call(
        _ln_body_chained,
        grid=(nsteps,),
        in_specs=[buf_spec, x_spec] + small_specs,
        out_specs=out_spec,
        out_shape=out_shape,
        input_output_aliases={0: 0},
        scratch_shapes=scratch,
    )(buf, tok, segment_emb, ln_w1, ln_b1)


def kernel(input_tensor, token_emb, segment_emb, ln_w, ln_b):
    bsz, sent_len = input_tensor.shape
    dim = token_emb.shape[1]
    n_rows = bsz * sent_len
    rows_c = n_rows // _K
    bsz_c = bsz // _K
    n_chunks = rows_c // (_NW * _CHUNK)
    idx4 = input_tensor.astype(jnp.int32).reshape(_K, _NW, n_chunks, _CHUNK)
    ln_w1 = ln_w.reshape(1, dim)
    ln_b1 = ln_b.reshape(1, dim)
    toks = [
        _sc_gather(token_emb, idx4[c]).reshape(bsz_c, sent_len, dim)
        for c in range(_K)
    ]
    buf = None
    for c in range(_K):
        buf = _tc_ln_chunk(buf, toks[c], segment_emb, ln_w1, ln_b1, c, bsz)
    return buf


# speedup vs baseline: 7.5153x; 1.0857x over previous
import functools
import math

import jax
import jax.numpy as jnp
from jax import lax
from jax.experimental import pallas as pl
from jax.experimental.pallas import tpu as pltpu
from jax.experimental.pallas import tpu_sc as plsc

_NC, _NS = 2, 16
_NW = _NC * _NS
_K = 8
_CHUNK = 80


def _sc_gather(token_emb, idx3):
    nw, n_chunks, ch = idx3.shape
    rows_total = nw * n_chunks * ch
    dim = token_emb.shape[1]
    mesh = plsc.VectorSubcoreMesh(core_axis_name="c", subcore_axis_name="s")

    assert n_chunks % 2 == 0
    nh = n_chunks // 2

    @functools.partial(
        pl.kernel,
        mesh=mesh,
        out_type=jax.ShapeDtypeStruct((rows_total, dim), jnp.float32),
        scratch_types=[
            pltpu.VMEM((n_chunks, ch), jnp.int32),
            pltpu.VMEM((ch, dim), jnp.float32),
            pltpu.VMEM((ch, dim), jnp.float32),
            pltpu.SemaphoreType.DMA,
            pltpu.SemaphoreType.DMA,
        ],
    )
    def gather_kernel(table_hbm, idx_hbm, out_hbm, idx_v, rows0, rows1,
                      gsem0, gsem1):
        wid = lax.axis_index("s") * _NC + lax.axis_index("c")
        pltpu.sync_copy(idx_hbm.at[wid], idx_v)
        base = wid * (n_chunks * ch)
        pltpu.async_copy(table_hbm.at[idx_v.at[0]], rows0, gsem0)

        def body(i, carry):
            j0 = 2 * i
            j1 = j0 + 1
            pltpu.async_copy(table_hbm.at[idx_v.at[j1]], rows1, gsem1)
            pltpu.make_async_copy(table_hbm.at[idx_v.at[j0]], rows0, gsem0).wait()
            pltpu.sync_copy(rows0, out_hbm.at[pl.ds(base + j0 * ch, ch)])

            @pl.when(i < nh - 1)
            def _():
                pltpu.async_copy(table_hbm.at[idx_v.at[j0 + 2]], rows0, gsem0)

            pltpu.make_async_copy(table_hbm.at[idx_v.at[j1]], rows1, gsem1).wait()
            pltpu.sync_copy(rows1, out_hbm.at[pl.ds(base + j1 * ch, ch)])
            return carry

        lax.fori_loop(0, nh, body, 0)

    return gather_kernel(token_emb, idx3)


def _make_const(const_ref, seg_ref):
    sent_len, dim = const_ref.shape
    row = lax.broadcasted_iota(jnp.int32, (sent_len, dim), 0)
    col = lax.broadcasted_iota(jnp.int32, (sent_len, dim), 1)
    pos = row.astype(jnp.float32)
    dexp = 2.0 * col.astype(jnp.float32) * (1.0 / dim)
    angle = pos * jnp.exp(-math.log(10000.0) * dexp)
    pe = jnp.where(col % 2 == 0, jnp.sin(angle), jnp.cos(angle))
    segc = jnp.where(row >= (sent_len // 2 + 1),
                     seg_ref[1, :][None, :], seg_ref[0, :][None, :])
    const_ref[...] = pe + segc


def _ln_compute(x_ref, w_ref, b_ref, out_ref, const_ref):
    y = x_ref[...] + const_ref[...][None]
    mean = jnp.mean(y, axis=-1, keepdims=True)
    yc = y - mean
    var = jnp.mean(yc * yc, axis=-1, keepdims=True)
    inv = lax.rsqrt(var + 1e-5)
    out_ref[...] = yc * inv * w_ref[...] + b_ref[...]


def _ln_body_first(x_ref, seg_ref, w_ref, b_ref, out_ref, const_ref):
    @pl.when(pl.program_id(0) == 0)
    def _():
        _make_const(const_ref, seg_ref)

    _ln_compute(x_ref, w_ref, b_ref, out_ref, const_ref)


def _ln_body_chained(buf_ref, x_ref, seg_ref, w_ref, b_ref, out_ref, const_ref):
    del buf_ref
    @pl.when(pl.program_id(0) == 0)
    def _():
        _make_const(const_ref, seg_ref)

    _ln_compute(x_ref, w_ref, b_ref, out_ref, const_ref)


def _tc_ln_chunk(buf, tok, segment_emb, ln_w1, ln_b1, chunk, bsz_total):
    bsz_c, sent_len, dim = tok.shape
    bb = 8
    nsteps = bsz_c // bb
    base = chunk * nsteps
    x_spec = pl.BlockSpec((bb, sent_len, dim), lambda i: (i, 0, 0))
    small_specs = [
        pl.BlockSpec((8, dim), lambda i: (0, 0)),
        pl.BlockSpec((1, dim), lambda i: (0, 0)),
        pl.BlockSpec((1, dim), lambda i: (0, 0)),
    ]
    out_spec = pl.BlockSpec((bb, sent_len, dim), lambda i: (base + i, 0, 0))
    out_shape = jax.ShapeDtypeStruct((bsz_total, sent_len, dim), jnp.float32)
    scratch = [pltpu.VMEM((sent_len, dim), jnp.float32)]
    if buf is None:
        return pl.pallas_call(
            _ln_body_first,
            grid=(nsteps,),
            in_specs=[x_spec] + small_specs,
            out_specs=out_spec,
            out_shape=out_shape,
            scratch_shapes=scratch,
        )(tok, segment_emb, ln_w1, ln_b1)
    buf_spec = pl.BlockSpec((1, 8, dim), lambda i: (0, 0, 0))
    return pl.pallas_call(
        _ln_body_chained,
        grid=(nsteps,),
        in_specs=[buf_spec, x_spec] + small_specs,
        out_specs=out_spec,
        out_shape=out_shape,
        input_output_aliases={0: 0},
        scratch_shapes=scratch,
    )(buf, tok, segment_emb, ln_w1, ln_b1)


def kernel(input_tensor, token_emb, segment_emb, ln_w, ln_b):
    bsz, sent_len = input_tensor.shape
    dim = token_emb.shape[1]
    n_rows = bsz * sent_len
    rows_c = n_rows // _K
    bsz_c = bsz // _K
    n_chunks = rows_c // (_NW * _CHUNK)
    idx4 = input_tensor.astype(jnp.int32).reshape(_K, _NW, n_chunks, _CHUNK)
    ln_w1 = ln_w.reshape(1, dim)
    ln_b1 = ln_b.reshape(1, dim)
    toks = [
        _sc_gather(token_emb, idx4[c]).reshape(bsz_c, sent_len, dim)
        for c in range(_K)
    ]
    buf = None
    for c in range(_K):
        buf = _tc_ln_chunk(buf, toks[c], segment_emb, ln_w1, ln_b1, c, bsz)
    return buf
